# baseline (speedup 1.0000x reference)
import functools
import math

import jax
import jax.numpy as jnp
from jax import lax
from jax.experimental import pallas as pl
from jax.experimental.pallas import tpu as pltpu
from jax.experimental.pallas import tpu_sc as plsc


def _scale_body(scale, t_ref, o_ref):
    o_ref[...] = t_ref[...] * scale


def _scale_table(table, scale):
    v, d = table.shape
    block = 2000
    assert v % block == 0
    return pl.pallas_call(
        functools.partial(_scale_body, scale),
        grid=(v // block,),
        in_specs=[pl.BlockSpec((block, d), lambda i: (i, 0))],
        out_specs=pl.BlockSpec((block, d), lambda i: (i, 0)),
        out_shape=jax.ShapeDtypeStruct((v, d), table.dtype),
    )(table)


@functools.lru_cache(maxsize=None)
def _make_gather(total, d):
    info = plsc.get_sparse_core_info()
    nc, ns = info.num_cores, info.num_subcores
    nw = nc * ns
    chunk = 128
    m = 2
    nbuf = 2
    per_w = total // nw
    n_chunks = per_w // chunk
    n_bufs_total = n_chunks // m
    n_groups = n_bufs_total // nbuf
    assert total % (nw * chunk * m * nbuf) == 0
    rows_per_buf = m * chunk
    mesh = plsc.VectorSubcoreMesh(core_axis_name="c", subcore_axis_name="s")

    @functools.partial(
        pl.kernel,
        mesh=mesh,
        out_type=jax.ShapeDtypeStruct((total, d), jnp.float32),
        scratch_types=[
            pltpu.VMEM((n_chunks, chunk), jnp.int32),
            pltpu.VMEM((nbuf, rows_per_buf, d), jnp.float32),
        ]
        + [pltpu.SemaphoreType.DMA] * (2 * nbuf),
    )
    def gather(idx_hbm, table_hbm, out_hbm, idx_v, rows_v, *sems):
        sg, so = sems[:nbuf], sems[nbuf:]
        wid = lax.axis_index("s") * nc + lax.axis_index("c")
        base = wid * per_w

        pltpu.sync_copy(idx_hbm.at[pl.ds(wid * n_chunks, n_chunks)], idx_v)

        def fire_gathers(i, b):
            for j in range(m):
                pltpu.async_copy(
                    table_hbm.at[idx_v.at[i * m + j]],
                    rows_v.at[b].at[pl.ds(j * chunk, chunk)],
                    sg[b],
                )

        def wait_gathers(b):
            pltpu.make_async_copy(
                out_hbm.at[pl.ds(0, rows_per_buf)], rows_v.at[b], sg[b]
            ).wait()

        def fire_out(i, b):
            pltpu.async_copy(
                rows_v.at[b],
                out_hbm.at[pl.ds(base + i * rows_per_buf, rows_per_buf)],
                so[b],
            )

        def wait_out(b):
            pltpu.make_async_copy(
                rows_v.at[b], out_hbm.at[pl.ds(0, rows_per_buf)], so[b]
            ).wait()

        for b in range(nbuf):
            fire_gathers(b, b)

        def body(g, carry):
            for b in range(nbuf):
                wait_gathers(b)
                fire_out(g * nbuf + b, b)
            for b in range(nbuf):
                wait_out(b)
                fire_gathers((g + 1) * nbuf + b, b)
            return carry

        lax.fori_loop(0, n_groups - 1, body, 0)

        last = (n_groups - 1) * nbuf
        for b in range(nbuf):
            wait_gathers(b)
            fire_out(last + b, b)
        for b in range(nbuf):
            wait_out(b)

    return gather


def kernel(tok_ids, emb_table):
    b, l = tok_ids.shape
    v, d = emb_table.shape
    scaled = _scale_table(emb_table, math.sqrt(float(d)))
    flat = tok_ids.reshape(-1, 128).astype(jnp.int32)
    out = _make_gather(b * l, d)(flat, scaled)
    return out.reshape(b, l, d)


# speedup vs baseline: 1.0517x; 1.0517x over previous
import functools
import math

import jax
import jax.numpy as jnp
from jax import lax
from jax.experimental import pallas as pl
from jax.experimental.pallas import tpu as pltpu
from jax.experimental.pallas import tpu_sc as plsc


def _scale_body(scale, t_ref, o_ref):
    o_ref[...] = t_ref[...] * scale


def _scale_table(table, scale):
    v, d = table.shape
    block = 10000
    assert v % block == 0
    return pl.pallas_call(
        functools.partial(_scale_body, scale),
        grid=(v // block,),
        in_specs=[pl.BlockSpec((block, d), lambda i: (i, 0))],
        out_specs=pl.BlockSpec((block, d), lambda i: (i, 0)),
        out_shape=jax.ShapeDtypeStruct((v, d), table.dtype),
    )(table)


@functools.lru_cache(maxsize=None)
def _make_gather(total, d):
    info = plsc.get_sparse_core_info()
    nc, ns = info.num_cores, info.num_subcores
    nw = nc * ns
    chunk = 128
    m = 2
    nbuf = 2
    per_w = total // nw
    n_chunks = per_w // chunk
    n_bufs_total = n_chunks // m
    n_groups = n_bufs_total // nbuf
    assert total % (nw * chunk * m * nbuf) == 0
    rows_per_buf = m * chunk
    mesh = plsc.VectorSubcoreMesh(core_axis_name="c", subcore_axis_name="s")

    @functools.partial(
        pl.kernel,
        mesh=mesh,
        out_type=jax.ShapeDtypeStruct((total, d), jnp.float32),
        scratch_types=[
            pltpu.VMEM((n_chunks, chunk), jnp.int32),
            pltpu.VMEM((nbuf, rows_per_buf, d), jnp.float32),
        ]
        + [pltpu.SemaphoreType.DMA] * (2 * nbuf),
    )
    def gather(idx_hbm, table_hbm, out_hbm, idx_v, rows_v, *sems):
        sg, so = sems[:nbuf], sems[nbuf:]
        wid = lax.axis_index("s") * nc + lax.axis_index("c")
        base = wid * per_w

        pltpu.sync_copy(idx_hbm.at[pl.ds(wid * n_chunks, n_chunks)], idx_v)

        def fire_gathers(i, b):
            for j in range(m):
                pltpu.async_copy(
                    table_hbm.at[idx_v.at[i * m + j]],
                    rows_v.at[b].at[pl.ds(j * chunk, chunk)],
                    sg[b],
                )

        def wait_gathers(b):
            pltpu.make_async_copy(
                out_hbm.at[pl.ds(0, rows_per_buf)], rows_v.at[b], sg[b]
            ).wait()

        def fire_out(i, b):
            pltpu.async_copy(
                rows_v.at[b],
                out_hbm.at[pl.ds(base + i * rows_per_buf, rows_per_buf)],
                so[b],
            )

        def wait_out(b):
            pltpu.make_async_copy(
                rows_v.at[b], out_hbm.at[pl.ds(0, rows_per_buf)], so[b]
            ).wait()

        for b in range(nbuf):
            fire_gathers(b, b)

        def body(g, carry):
            for b in range(nbuf):
                wait_gathers(b)
                fire_out(g * nbuf + b, b)
            for b in range(nbuf):
                wait_out(b)
                fire_gathers((g + 1) * nbuf + b, b)
            return carry

        lax.fori_loop(0, n_groups - 1, body, 0)

        last = (n_groups - 1) * nbuf
        for b in range(nbuf):
            wait_gathers(b)
            fire_out(last + b, b)
        for b in range(nbuf):
            wait_out(b)

    return gather


def kernel(tok_ids, emb_table):
    b, l = tok_ids.shape
    v, d = emb_table.shape
    scaled = _scale_table(emb_table, math.sqrt(float(d)))
    flat = tok_ids.reshape(-1, 128).astype(jnp.int32)
    out = _make_gather(b * l, d)(flat, scaled)
    return out.reshape(b, l, d)


# speedup vs baseline: 1.0534x; 1.0017x over previous
import functools
import math

import jax
import jax.numpy as jnp
from jax import lax
from jax.experimental import pallas as pl
from jax.experimental.pallas import tpu as pltpu
from jax.experimental.pallas import tpu_sc as plsc


def _scale_body(scale, t_ref, o_ref):
    o_ref[...] = t_ref[...] * scale


def _scale_table(table, scale):
    v, d = table.shape
    block = 25000
    assert v % block == 0
    return pl.pallas_call(
        functools.partial(_scale_body, scale),
        grid=(v // block,),
        in_specs=[pl.BlockSpec((block, d), lambda i: (i, 0))],
        out_specs=pl.BlockSpec((block, d), lambda i: (i, 0)),
        out_shape=jax.ShapeDtypeStruct((v, d), table.dtype),
    )(table)


@functools.lru_cache(maxsize=None)
def _make_gather(total, d):
    info = plsc.get_sparse_core_info()
    nc, ns = info.num_cores, info.num_subcores
    nw = nc * ns
    chunk = 128
    m = 2
    nbuf = 2
    per_w = total // nw
    n_chunks = per_w // chunk
    n_bufs_total = n_chunks // m
    n_groups = n_bufs_total // nbuf
    assert total % (nw * chunk * m * nbuf) == 0
    rows_per_buf = m * chunk
    mesh = plsc.VectorSubcoreMesh(core_axis_name="c", subcore_axis_name="s")

    @functools.partial(
        pl.kernel,
        mesh=mesh,
        out_type=jax.ShapeDtypeStruct((total, d), jnp.float32),
        scratch_types=[
            pltpu.VMEM((n_chunks, chunk), jnp.int32),
            pltpu.VMEM((nbuf, rows_per_buf, d), jnp.float32),
        ]
        + [pltpu.SemaphoreType.DMA] * (2 * nbuf),
    )
    def gather(idx_hbm, table_hbm, out_hbm, idx_v, rows_v, *sems):
        sg, so = sems[:nbuf], sems[nbuf:]
        wid = lax.axis_index("s") * nc + lax.axis_index("c")
        base = wid * per_w

        pltpu.sync_copy(idx_hbm.at[pl.ds(wid * n_chunks, n_chunks)], idx_v)

        def fire_gathers(i, b):
            for j in range(m):
                pltpu.async_copy(
                    table_hbm.at[idx_v.at[i * m + j]],
                    rows_v.at[b].at[pl.ds(j * chunk, chunk)],
                    sg[b],
                )

        def wait_gathers(b):
            pltpu.make_async_copy(
                out_hbm.at[pl.ds(0, rows_per_buf)], rows_v.at[b], sg[b]
            ).wait()

        def fire_out(i, b):
            pltpu.async_copy(
                rows_v.at[b],
                out_hbm.at[pl.ds(base + i * rows_per_buf, rows_per_buf)],
                so[b],
            )

        def wait_out(b):
            pltpu.make_async_copy(
                rows_v.at[b], out_hbm.at[pl.ds(0, rows_per_buf)], so[b]
            ).wait()

        for b in range(nbuf):
            fire_gathers(b, b)

        def body(g, carry):
            for b in range(nbuf):
                wait_gathers(b)
                fire_out(g * nbuf + b, b)
            for b in range(nbuf):
                wait_out(b)
                fire_gathers((g + 1) * nbuf + b, b)
            return carry

        lax.fori_loop(0, n_groups - 1, body, 0)

        last = (n_groups - 1) * nbuf
        for b in range(nbuf):
            wait_gathers(b)
            fire_out(last + b, b)
        for b in range(nbuf):
            wait_out(b)

    return gather


def kernel(tok_ids, emb_table):
    b, l = tok_ids.shape
    v, d = emb_table.shape
    scaled = _scale_table(emb_table, math.sqrt(float(d)))
    flat = tok_ids.reshape(-1, 128).astype(jnp.int32)
    out = _make_gather(b * l, d)(flat, scaled)
    return out.reshape(b, l, d)


# speedup vs baseline: 1.0773x; 1.0227x over previous
import functools
import math

import jax
import jax.numpy as jnp
from jax import lax
from jax.experimental import pallas as pl
from jax.experimental.pallas import tpu as pltpu
from jax.experimental.pallas import tpu_sc as plsc


def _scale_body(scale, t_ref, o_ref):
    o_ref[...] = t_ref[...] * scale


def _scale_table(table, scale):
    v, d = table.shape
    block = 25000
    assert v % block == 0
    return pl.pallas_call(
        functools.partial(_scale_body, scale),
        grid=(v // block,),
        in_specs=[pl.BlockSpec((block, d), lambda i: (i, 0))],
        out_specs=pl.BlockSpec((block, d), lambda i: (i, 0)),
        out_shape=jax.ShapeDtypeStruct((v, d), table.dtype),
    )(table)


@functools.lru_cache(maxsize=None)
def _make_gather(total, d):
    info = plsc.get_sparse_core_info()
    nc, ns = info.num_cores, info.num_subcores
    nw = nc * ns
    chunk = 80
    m = 2
    nbuf = 4
    per_w = total // nw
    n_chunks = per_w // chunk
    n_bufs_total = n_chunks // m
    n_groups = n_bufs_total // nbuf
    assert total % (nw * chunk * m * nbuf) == 0
    rows_per_buf = m * chunk
    mesh = plsc.VectorSubcoreMesh(core_axis_name="c", subcore_axis_name="s")

    @functools.partial(
        pl.kernel,
        mesh=mesh,
        out_type=jax.ShapeDtypeStruct((total, d), jnp.float32),
        scratch_types=[
            pltpu.VMEM((n_chunks, chunk), jnp.int32),
            pltpu.VMEM((nbuf, rows_per_buf, d), jnp.float32),
        ]
        + [pltpu.SemaphoreType.DMA] * (2 * nbuf),
    )
    def gather(idx_hbm, table_hbm, out_hbm, idx_v, rows_v, *sems):
        sg, so = sems[:nbuf], sems[nbuf:]
        wid = lax.axis_index("s") * nc + lax.axis_index("c")
        base = wid * per_w

        pltpu.sync_copy(idx_hbm.at[pl.ds(wid * n_chunks, n_chunks)], idx_v)

        def fire_gathers(i, b):
            for j in range(m):
                pltpu.async_copy(
                    table_hbm.at[idx_v.at[i * m + j]],
                    rows_v.at[b].at[pl.ds(j * chunk, chunk)],
                    sg[b],
                )

        def wait_gathers(b):
            pltpu.make_async_copy(
                out_hbm.at[pl.ds(0, rows_per_buf)], rows_v.at[b], sg[b]
            ).wait()

        def fire_out(i, b):
            pltpu.async_copy(
                rows_v.at[b],
                out_hbm.at[pl.ds(base + i * rows_per_buf, rows_per_buf)],
                so[b],
            )

        def wait_out(b):
            pltpu.make_async_copy(
                rows_v.at[b], out_hbm.at[pl.ds(0, rows_per_buf)], so[b]
            ).wait()

        for b in range(nbuf):
            fire_gathers(b, b)

        def body(g, carry):
            for b in range(nbuf):
                wait_gathers(b)
                fire_out(g * nbuf + b, b)
            for b in range(nbuf):
                wait_out(b)
                fire_gathers((g + 1) * nbuf + b, b)
            return carry

        lax.fori_loop(0, n_groups - 1, body, 0)

        last = (n_groups - 1) * nbuf
        for b in range(nbuf):
            wait_gathers(b)
            fire_out(last + b, b)
        for b in range(nbuf):
            wait_out(b)

    return gather


def kernel(tok_ids, emb_table):
    b, l = tok_ids.shape
    v, d = emb_table.shape
    scaled = _scale_table(emb_table, math.sqrt(float(d)))
    flat = tok_ids.reshape(-1, 80).astype(jnp.int32)
    out = _make_gather(b * l, d)(flat, scaled)
    return out.reshape(b, l, d)
